# baseline (speedup 1.0000x reference)
import functools

import jax
import jax.numpy as jnp
from jax import lax
from jax.experimental import pallas as pl
from jax.experimental.pallas import tpu as pltpu
from jax.experimental.pallas import tpu_sc as plsc

B = 16384
F = 26
FIELD = 10000
NC = 2
NS = 16
LANES = 16
HALF = B // NC
FPT = 2
ACT = F // FPT
ROWS_OUT = HALF // NS
NCH = 2
CHUNK = HALF // NCH
TPC = NS // NCH

_mesh = plsc.VectorSubcoreMesh(
    core_axis_name="c", subcore_axis_name="s", num_cores=NC, num_subcores=NS
)


@functools.partial(
    pl.kernel,
    out_type=jax.ShapeDtypeStruct((B,), jnp.float32),
    mesh=_mesh,
    compiler_params=pltpu.CompilerParams(needs_layout_passes=False),
    scratch_types=[
        pltpu.VMEM((FPT * FIELD,), jnp.float32),
        pltpu.VMEM((1, HALF), jnp.int32),
        pltpu.VMEM((1, HALF), jnp.int32),
        pltpu.VMEM((HALF,), jnp.float32),
        pltpu.VMEM((ACT * ROWS_OUT,), jnp.float32),
        pltpu.VMEM((ROWS_OUT,), jnp.float32),
        pltpu.VMEM((LANES,), jnp.float32),
        pltpu.VMEM_SHARED((NS * ACT * ROWS_OUT,), jnp.float32),
        pltpu.SemaphoreType.DMA,
        pltpu.SemaphoreType.DMA((NCH,)),
        pltpu.SemaphoreType.DMA,
        pltpu.SemaphoreType.DMA,
        pltpu.SemaphoreType.DMA,
    ],
)
def _lr_kernel(
    xt_hbm, tbl_hbm, bias_hbm, out_hbm,
    tbl_v, xc0_v, xc1_v, acc_v, part_v, outb_v, bias_v, shared,
    sem_t, sem_x, sem_p, sem_r, sem_b,
):
    c = lax.axis_index("c")
    s = lax.axis_index("s")
    base_b = pl.multiple_of(c * HALF, HALF)
    cp_b = pltpu.async_copy(bias_hbm, bias_v.at[pl.ds(0, 1)], sem_b)

    @pl.when(s < ACT)
    def _gather_phase():
        f0 = 2 * s
        tstart = pl.multiple_of(f0 * FIELD, 2 * FIELD)
        cp_t = pltpu.async_copy(tbl_hbm.at[pl.ds(tstart, FPT * FIELD)], tbl_v, sem_t)
        cpx = []
        for ch in range(NCH):
            lo = ch * CHUNK
            cpx.append(
                pltpu.async_copy(
                    xt_hbm.at[pl.ds(f0, 1), pl.ds(base_b + lo, CHUNK)],
                    xc0_v.at[:, pl.ds(lo, CHUNK)],
                    sem_x.at[ch],
                )
            )
            cpx.append(
                pltpu.async_copy(
                    xt_hbm.at[pl.ds(f0 + 1, 1), pl.ds(base_b + lo, CHUNK)],
                    xc1_v.at[:, pl.ds(lo, CHUNK)],
                    sem_x.at[ch],
                )
            )
        cp_t.wait()

        pubs = []
        for ch in range(NCH):
            lo = ch * CHUNK
            cpx[2 * ch].wait()
            cpx[2 * ch + 1].wait()

            @plsc.parallel_loop(lo, lo + CHUNK, LANES, unroll=1)
            def _gather_loop(i):
                o = pl.multiple_of(i, LANES)
                i0 = xc0_v[0, pl.ds(o, LANES)]
                i1 = xc1_v[0, pl.ds(o, LANES)] + FIELD
                v = plsc.load_gather(tbl_v, [i0]) + plsc.load_gather(tbl_v, [i1])
                acc_v[pl.ds(o, LANES)] = v

            for k in range(TPC):
                t_out = ch * TPC + k
                dst = pl.multiple_of((t_out * ACT + s) * ROWS_OUT, ROWS_OUT)
                pubs.append(
                    pltpu.async_copy(
                        acc_v.at[pl.ds(t_out * ROWS_OUT, ROWS_OUT)],
                        shared.at[pl.ds(dst, ROWS_OUT)],
                        sem_p,
                    )
                )
        for cp in pubs:
            cp.wait()

    plsc.subcore_barrier()

    src = pl.multiple_of(s * ACT * ROWS_OUT, ROWS_OUT)
    pltpu.async_copy(shared.at[pl.ds(src, ACT * ROWS_OUT)], part_v, sem_r).wait()
    cp_b.wait()
    bv = bias_v[...]
    bias_vec = lax.broadcast_in_dim(
        jnp.sum(jnp.where(lax.iota(jnp.int32, LANES) == 0, bv, 0.0)), (LANES,), ()
    )

    @plsc.parallel_loop(0, ROWS_OUT, LANES, unroll=1)
    def _reduce_loop(k):
        o = pl.multiple_of(k, LANES)
        a = part_v[pl.ds(o, LANES)]
        for j in range(1, ACT):
            a = a + part_v[pl.ds(j * ROWS_OUT + o, LANES)]
        z = a + bias_vec
        outb_v[pl.ds(o, LANES)] = 1.0 / (1.0 + jnp.exp(-z))

    off = pl.multiple_of(s * ROWS_OUT, ROWS_OUT)
    pltpu.sync_copy(outb_v, out_hbm.at[pl.ds(base_b + off, ROWS_OUT)])


def kernel(x, table, bias):
    xt = jnp.transpose(x.astype(jnp.int32))
    tbl = table.reshape(-1)
    out = _lr_kernel(xt, tbl, bias.astype(jnp.float32))
    return out.reshape(B, 1)


# speedup vs baseline: 1.0423x; 1.0423x over previous
import functools

import jax
import jax.numpy as jnp
from jax import lax
from jax.experimental import pallas as pl
from jax.experimental.pallas import tpu as pltpu
from jax.experimental.pallas import tpu_sc as plsc

B = 16384
F = 26
FIELD = 10000
NC = 2
NS = 16
LANES = 16
HALF = B // NC
FPT = 2
ACT = F // FPT
ROWS_OUT = HALF // NS
NCH = 2
CHUNK = HALF // NCH
TPC = NS // NCH

_mesh = plsc.VectorSubcoreMesh(
    core_axis_name="c", subcore_axis_name="s", num_cores=NC, num_subcores=NS
)


@functools.partial(
    pl.kernel,
    out_type=jax.ShapeDtypeStruct((B,), jnp.float32),
    mesh=_mesh,
    compiler_params=pltpu.CompilerParams(needs_layout_passes=False),
    scratch_types=[
        pltpu.VMEM((FPT * FIELD,), jnp.float32),
        pltpu.VMEM((1, HALF), jnp.int32),
        pltpu.VMEM((1, HALF), jnp.int32),
        pltpu.VMEM((HALF,), jnp.float32),
        pltpu.VMEM((ACT * ROWS_OUT,), jnp.float32),
        pltpu.VMEM((ROWS_OUT,), jnp.float32),
        pltpu.VMEM((LANES,), jnp.float32),
        pltpu.VMEM_SHARED((NS * ACT * ROWS_OUT,), jnp.float32),
        pltpu.SemaphoreType.DMA,
        pltpu.SemaphoreType.DMA((NCH,)),
        pltpu.SemaphoreType.DMA,
        pltpu.SemaphoreType.DMA,
        pltpu.SemaphoreType.DMA,
    ],
)
def _lr_kernel(
    xt_hbm, tbl_hbm, bias_hbm, out_hbm,
    tbl_v, xc0_v, xc1_v, acc_v, part_v, outb_v, bias_v, shared,
    sem_t, sem_x, sem_p, sem_r, sem_b,
):
    c = lax.axis_index("c")
    s = lax.axis_index("s")
    base_b = pl.multiple_of(c * HALF, HALF)
    cp_b = pltpu.async_copy(bias_hbm, bias_v.at[pl.ds(0, 1)], sem_b)

    @pl.when(s < ACT)
    def _gather_phase():
        f0 = 2 * s
        tstart = pl.multiple_of(f0 * FIELD, 2 * FIELD)
        cp_t = pltpu.async_copy(tbl_hbm.at[pl.ds(tstart, FPT * FIELD)], tbl_v, sem_t)
        cpx = []
        for ch in range(NCH):
            lo = ch * CHUNK
            cpx.append(
                pltpu.async_copy(
                    xt_hbm.at[pl.ds(f0, 1), pl.ds(base_b + lo, CHUNK)],
                    xc0_v.at[:, pl.ds(lo, CHUNK)],
                    sem_x.at[ch],
                )
            )
            cpx.append(
                pltpu.async_copy(
                    xt_hbm.at[pl.ds(f0 + 1, 1), pl.ds(base_b + lo, CHUNK)],
                    xc1_v.at[:, pl.ds(lo, CHUNK)],
                    sem_x.at[ch],
                )
            )
        cp_t.wait()

        pubs = []
        for ch in range(NCH):
            lo = ch * CHUNK
            cpx[2 * ch].wait()
            cpx[2 * ch + 1].wait()

            @plsc.parallel_loop(lo, lo + CHUNK, LANES, unroll=4)
            def _gather_loop(i):
                o = pl.multiple_of(i, LANES)
                i0 = xc0_v[0, pl.ds(o, LANES)]
                i1 = xc1_v[0, pl.ds(o, LANES)] + FIELD
                v = plsc.load_gather(tbl_v, [i0]) + plsc.load_gather(tbl_v, [i1])
                acc_v[pl.ds(o, LANES)] = v

            for k in range(TPC):
                t_out = ch * TPC + k
                dst = pl.multiple_of((t_out * ACT + s) * ROWS_OUT, ROWS_OUT)
                pubs.append(
                    pltpu.async_copy(
                        acc_v.at[pl.ds(t_out * ROWS_OUT, ROWS_OUT)],
                        shared.at[pl.ds(dst, ROWS_OUT)],
                        sem_p,
                    )
                )
        for cp in pubs:
            cp.wait()

    plsc.subcore_barrier()

    src = pl.multiple_of(s * ACT * ROWS_OUT, ROWS_OUT)
    pltpu.async_copy(shared.at[pl.ds(src, ACT * ROWS_OUT)], part_v, sem_r).wait()
    cp_b.wait()
    bv = bias_v[...]
    bias_vec = lax.broadcast_in_dim(
        jnp.sum(jnp.where(lax.iota(jnp.int32, LANES) == 0, bv, 0.0)), (LANES,), ()
    )

    @plsc.parallel_loop(0, ROWS_OUT, LANES, unroll=1)
    def _reduce_loop(k):
        o = pl.multiple_of(k, LANES)
        a = part_v[pl.ds(o, LANES)]
        for j in range(1, ACT):
            a = a + part_v[pl.ds(j * ROWS_OUT + o, LANES)]
        z = a + bias_vec
        outb_v[pl.ds(o, LANES)] = 1.0 / (1.0 + jnp.exp(-z))

    off = pl.multiple_of(s * ROWS_OUT, ROWS_OUT)
    pltpu.sync_copy(outb_v, out_hbm.at[pl.ds(base_b + off, ROWS_OUT)])


def kernel(x, table, bias):
    xt = jnp.transpose(x.astype(jnp.int32))
    tbl = table.reshape(-1)
    out = _lr_kernel(xt, tbl, bias.astype(jnp.float32))
    return out.reshape(B, 1)
